# Initial kernel scaffold; baseline (speedup 1.0000x reference)
import jax
import jax.numpy as jnp
from jax.experimental import pallas as pl


def kernel(x, Wf, bf, Wv, bv, Wp, bp, sim_alpha, sim_beta):
    raise NotImplementedError("write your pallas kernel here")


# speedup vs baseline: 1.9293x; 1.9293x over previous
import jax
import jax.numpy as jnp
from jax.experimental import pallas as pl

HEADS = 8
HD = 48
FOLD = 2
PW = 4
C = HEADS * HD
N = 1024
M = FOLD * FOLD * PW * PW

_BF = jnp.bfloat16
_F32 = jnp.float32
_HI = jax.lax.Precision.HIGHEST


def _bdot(a, b):
    return jnp.dot(a.astype(_BF), b.astype(_BF), preferred_element_type=_F32)


def _cluster_kernel(x_ref, wf_ref, bf_ref, wv_ref, bv_ref, wp_ref, bp_ref,
                    ab_ref, out_ref):
    xmat = x_ref[0]
    xf = _bdot(wf_ref[...], xmat) + bf_ref[...]
    val = _bdot(wv_ref[...], xmat) + bv_ref[...]

    ab = ab_ref[...]
    alpha = ab[:, 0:1]
    beta = ab[:, 1:2]

    n_iota = jax.lax.broadcasted_iota(jnp.int32, (N, M), 0)
    m_iota = jax.lax.broadcasted_iota(jnp.int32, (N, M), 1)
    w = n_iota // 32
    h = n_iota % 32
    quad = (w // 16) * 2 + (h // 16)
    m_of_n = quad * 16 + ((w % 16) // 4) * 4 + ((h % 16) // 4)
    pool = jnp.where(m_iota == m_of_n, 1.0 / 16.0, 0.0).astype(_F32)
    valid_nm = (m_iota // 16) == quad
    ones_row = jnp.full((1, N), 1.0, dtype=_F32)

    outs = []
    for e in range(HEADS):
        xf_h = jax.lax.slice(xf, (e * HD, 0), ((e + 1) * HD, N))
        v_h = jax.lax.slice(val, (e * HD, 0), ((e + 1) * HD, N))

        cen = jnp.dot(xf_h, pool, preferred_element_type=_F32, precision=_HI)
        vc = jnp.dot(v_h, pool, preferred_element_type=_F32, precision=_HI)

        cen_n = cen / jnp.maximum(
            jnp.sqrt(jnp.sum(cen * cen, axis=0, keepdims=True)), 1e-12)
        xf_n = xf_h / jnp.maximum(
            jnp.sqrt(jnp.sum(xf_h * xf_h, axis=0, keepdims=True)), 1e-12)

        sim = jax.nn.sigmoid(
            beta + alpha * jnp.einsum('cm,cn->mn',
                                      cen_n.astype(_BF), xf_n.astype(_BF),
                                      preferred_element_type=_F32))

        valid = jnp.transpose(valid_nm)
        simv = jnp.where(valid, sim, -1.0)
        amax = jnp.max(simv, axis=0, keepdims=True)
        mi = jax.lax.broadcasted_iota(jnp.int32, (M, N), 0)
        first = jnp.min(jnp.where(simv >= amax, mi, M), axis=0, keepdims=True)
        simm = jnp.where(mi == first, sim, 0.0)

        agg = jnp.einsum('cn,mn->cm', v_h.astype(_BF), simm.astype(_BF),
                         preferred_element_type=_F32) + vc
        denom = jnp.einsum('xn,mn->xm', ones_row, simm,
                           preferred_element_type=_F32, precision=_HI)
        out_m = agg / (denom + 1.0)
        out_h = _bdot(out_m, simm)
        outs.append(out_h)

    merged = jnp.concatenate(outs, axis=0)
    out_ref[0] = _bdot(wp_ref[...], merged) + bp_ref[...]


def kernel(x, Wf, bf, Wv, bv, Wp, bp, sim_alpha, sim_beta):
    B = x.shape[0]
    x2 = x.reshape(B, C, N)
    ab = jnp.concatenate([sim_alpha, sim_beta]).reshape(1, 2)
    bf2 = bf.reshape(C, 1)
    bv2 = bv.reshape(C, 1)
    bp2 = bp.reshape(C, 1)

    out = pl.pallas_call(
        _cluster_kernel,
        grid=(B,),
        in_specs=[
            pl.BlockSpec((1, C, N), lambda b: (b, 0, 0)),
            pl.BlockSpec((C, C), lambda b: (0, 0)),
            pl.BlockSpec((C, 1), lambda b: (0, 0)),
            pl.BlockSpec((C, C), lambda b: (0, 0)),
            pl.BlockSpec((C, 1), lambda b: (0, 0)),
            pl.BlockSpec((C, C), lambda b: (0, 0)),
            pl.BlockSpec((C, 1), lambda b: (0, 0)),
            pl.BlockSpec((1, 2), lambda b: (0, 0)),
        ],
        out_specs=pl.BlockSpec((1, C, N), lambda b: (b, 0, 0)),
        out_shape=jax.ShapeDtypeStruct((B, C, N), jnp.float32),
    )(x2, Wf, bf2, Wv, bv2, Wp, bp2, ab)

    return out.reshape(B, C, 32, 32)


# speedup vs baseline: 2.6639x; 1.3808x over previous
import jax
import jax.numpy as jnp
from jax.experimental import pallas as pl

HEADS = 8
HD = 48
FOLD = 2
PW = 4
C = HEADS * HD
N = 1024
M = FOLD * FOLD * PW * PW

_BF = jnp.bfloat16
_F32 = jnp.float32
_HI = jax.lax.Precision.HIGHEST


def _bdot(a, b):
    return jnp.dot(a.astype(_BF), b.astype(_BF), preferred_element_type=_F32)


def _cluster_kernel(x_ref, wf_ref, bf_ref, wv_ref, bv_ref, wp_ref, bp_ref,
                    ab_ref, out_ref):
    xmat = x_ref[0]
    xf = _bdot(wf_ref[...], xmat) + bf_ref[...]
    val = _bdot(wv_ref[...], xmat) + bv_ref[...]

    ab = ab_ref[...]
    alpha = ab[:, 0:1]
    beta = ab[:, 1:2]

    n_iota = jax.lax.broadcasted_iota(jnp.int32, (N, M), 0)
    m_iota = jax.lax.broadcasted_iota(jnp.int32, (N, M), 1)
    w = n_iota // 32
    h = n_iota % 32
    quad = (w // 16) * 2 + (h // 16)
    m_of_n = quad * 16 + ((w % 16) // 4) * 4 + ((h % 16) // 4)
    pool = jnp.where(m_iota == m_of_n, 1.0 / 16.0, 0.0).astype(_F32)
    valid_nm = (m_iota // 16) == quad
    valid = jnp.transpose(valid_nm)
    ones_row = jnp.full((1, N), 1.0, dtype=_F32)

    cen_all = jnp.dot(xf, pool, preferred_element_type=_F32, precision=_HI)
    vc_all = jnp.dot(val, pool, preferred_element_type=_F32, precision=_HI)

    xf3 = xf.reshape(HEADS, HD, N)
    xfn3 = xf3 / jnp.maximum(
        jnp.sqrt(jnp.sum(xf3 * xf3, axis=1, keepdims=True)), 1e-12)
    cen3 = cen_all.reshape(HEADS, HD, M)
    cenn3 = cen3 / jnp.maximum(
        jnp.sqrt(jnp.sum(cen3 * cen3, axis=1, keepdims=True)), 1e-12)

    outs = []
    for e in range(HEADS):
        v_h = jax.lax.slice(val, (e * HD, 0), ((e + 1) * HD, N))
        vc = jax.lax.slice(vc_all, (e * HD, 0), ((e + 1) * HD, M))
        xf_n = xfn3[e]
        cen_n = cenn3[e]

        sim = jax.nn.sigmoid(
            beta + alpha * jnp.einsum('cm,cn->mn',
                                      cen_n.astype(_BF), xf_n.astype(_BF),
                                      preferred_element_type=_F32))

        simv = jnp.where(valid, sim, -1.0)
        amax = jnp.max(simv, axis=0, keepdims=True)
        mi = jax.lax.broadcasted_iota(jnp.int32, (M, N), 0)
        first = jnp.min(jnp.where(simv >= amax, mi, M), axis=0, keepdims=True)
        simm = jnp.where(mi == first, sim, 0.0)

        v_aug = jnp.concatenate([v_h, ones_row], axis=0)
        agg_aug = jnp.einsum('cn,mn->cm', v_aug.astype(_BF), simm.astype(_BF),
                             preferred_element_type=_F32)
        agg = jax.lax.slice(agg_aug, (0, 0), (HD, M)) + vc
        denom = jax.lax.slice(agg_aug, (HD, 0), (HD + 1, M))
        out_m = agg / (denom + 1.0)
        out_h = _bdot(out_m, simm)
        outs.append(out_h)

    merged = jnp.concatenate(outs, axis=0)
    out_ref[0] = _bdot(wp_ref[...], merged) + bp_ref[...]


def kernel(x, Wf, bf, Wv, bv, Wp, bp, sim_alpha, sim_beta):
    B = x.shape[0]
    x2 = x.reshape(B, C, N)
    ab = jnp.concatenate([sim_alpha, sim_beta]).reshape(1, 2)
    bf2 = bf.reshape(C, 1)
    bv2 = bv.reshape(C, 1)
    bp2 = bp.reshape(C, 1)

    out = pl.pallas_call(
        _cluster_kernel,
        grid=(B,),
        in_specs=[
            pl.BlockSpec((1, C, N), lambda b: (b, 0, 0)),
            pl.BlockSpec((C, C), lambda b: (0, 0)),
            pl.BlockSpec((C, 1), lambda b: (0, 0)),
            pl.BlockSpec((C, C), lambda b: (0, 0)),
            pl.BlockSpec((C, 1), lambda b: (0, 0)),
            pl.BlockSpec((C, C), lambda b: (0, 0)),
            pl.BlockSpec((C, 1), lambda b: (0, 0)),
            pl.BlockSpec((1, 2), lambda b: (0, 0)),
        ],
        out_specs=pl.BlockSpec((1, C, N), lambda b: (b, 0, 0)),
        out_shape=jax.ShapeDtypeStruct((B, C, N), jnp.float32),
    )(x2, Wf, bf2, Wv, bv2, Wp, bp2, ab)

    return out.reshape(B, C, 32, 32)
